# baseline (speedup 1.0000x reference)
import functools

import jax
import jax.numpy as jnp
import numpy as np
from jax.experimental import pallas as pl
from jax.experimental.pallas import tpu as pltpu

_BASE_K = 64
_MIN_K = 16
_MAX_K = 512

_CHUNK = 256


def _count_true(m):
    v = jnp.where(m, jnp.int16(1), jnp.int16(0))
    w = v.shape[1]
    while w > 256:
        v = v[:, : w // 2] + v[:, w // 2:]
        w //= 2
    return jnp.sum(v.astype(jnp.int32), axis=1, keepdims=True)


def _select_body(k_fixed, i, sref, mask_ref, s):
    rows, width = sref.shape[1], sref.shape[2]
    r0 = i * rows
    x = sref[0]

    b = jax.lax.bitcast_convert_type(x, jnp.int32)
    sk = b ^ ((b >> 31) & jnp.int32(0x7FFFFFFF))

    q = r0 + jax.lax.broadcasted_iota(jnp.int32, (rows, 1), 0)
    if r0 + 1 >= k_fixed:
        k_eff = jnp.full((rows, 1), k_fixed, jnp.int32)
    else:
        k_eff = jnp.minimum(jnp.int32(k_fixed), q + 1)

    c = jax.lax.broadcasted_iota(jnp.int32, (rows, width), 1)
    hi = jnp.where(c <= q, sk >> 16, jnp.int32(-32768)).astype(jnp.int16)
    lo = ((sk & jnp.int32(0xFFFF)) - jnp.int32(32768)).astype(jnp.int16)

    def _thresh16(cand):
        return (cand - jnp.int32(32768)).astype(jnp.int16)

    def body_hi(it, t):
        cand = t | (jnp.int32(1) << (15 - it))
        cnt = _count_true(hi >= _thresh16(cand))
        return jnp.where(cnt >= k_eff, cand, t)

    t_hi = jax.lax.fori_loop(0, 16, body_hi,
                             jnp.zeros((rows, 1), jnp.int32), unroll=8)
    h_s = _thresh16(t_hi)

    gt = hi > h_s
    eq = hi == h_s
    k_rem = k_eff - _count_true(gt)
    mlo = jnp.where(eq, lo, jnp.int16(-(2**15)))

    def body_lo(it, t):
        cand = t | (jnp.int32(1) << (15 - it))
        cnt = _count_true(mlo >= _thresh16(cand))
        return jnp.where(cnt >= k_rem, cand, t)

    t_lo = jax.lax.fori_loop(0, 16, body_lo,
                             jnp.zeros((rows, 1), jnp.int32), unroll=8)
    l_s = _thresh16(t_lo)

    m = gt | (eq & (lo >= l_s))
    if width < s:
        m = jnp.concatenate(
            [m, jnp.zeros((rows, s - width), jnp.bool_)], axis=1)
    mask_ref[0] = m


def _mask_kernel(k_fixed, nchunks, *refs):
    mask_ref = refs[-1]
    s = mask_ref.shape[2]
    j = pl.program_id(1)

    for i in range(nchunks):

        @pl.when(j == i)
        def _(i=i):
            _select_body(k_fixed, i, refs[i], mask_ref, s)


@functools.partial(jax.jit, static_argnames=())
def kernel(x, index_scores, Wq, Wk):
    bsz, s, _ = index_scores.shape
    k_fixed = min(_BASE_K, s)
    k_fixed = int(np.clip(k_fixed, _MIN_K, min(_MAX_K, s)))

    rows = min(_CHUNK, s)
    nchunks = s // rows
    grid = (bsz, nchunks)
    in_specs = [
        pl.BlockSpec((1, rows, (i + 1) * rows),
                     functools.partial(lambda i, b, r: (b, i, 0), i))
        for i in range(nchunks)
    ]
    mask = pl.pallas_call(
        functools.partial(_mask_kernel, k_fixed, nchunks),
        grid=grid,
        in_specs=in_specs,
        out_specs=pl.BlockSpec((1, rows, s), lambda b, r: (b, r, 0)),
        out_shape=jax.ShapeDtypeStruct((bsz, s, s), jnp.bool_),
        compiler_params=pltpu.CompilerParams(
            dimension_semantics=("parallel", "parallel")),
    )(*([index_scores] * nchunks))

    k_values = jnp.full((bsz, s), k_fixed, dtype=jnp.int32)
    return (mask, k_values)


# speedup vs baseline: 1.2124x; 1.2124x over previous
import functools

import jax
import jax.numpy as jnp
import numpy as np
from jax.experimental import pallas as pl
from jax.experimental.pallas import tpu as pltpu

_BASE_K = 64
_MIN_K = 16
_MAX_K = 512

_CHUNK = 512


def _count_true(m):
    v = jnp.where(m, jnp.int16(1), jnp.int16(0))
    w = v.shape[1]
    while w > 256:
        v = v[:, : w // 2] + v[:, w // 2:]
        w //= 2
    return jnp.sum(v.astype(jnp.int32), axis=1, keepdims=True)


def _select_body(k_fixed, i, sref, mask_ref, s):
    rows, width = sref.shape[1], sref.shape[2]
    r0 = i * rows
    x = sref[0]

    b = jax.lax.bitcast_convert_type(x, jnp.int32)
    sk = b ^ ((b >> 31) & jnp.int32(0x7FFFFFFF))

    q = r0 + jax.lax.broadcasted_iota(jnp.int32, (rows, 1), 0)
    if r0 + 1 >= k_fixed:
        k_eff = jnp.full((rows, 1), k_fixed, jnp.int32)
    else:
        k_eff = jnp.minimum(jnp.int32(k_fixed), q + 1)

    c = jax.lax.broadcasted_iota(jnp.int32, (rows, width), 1)
    hi = jnp.where(c <= q, sk >> 16, jnp.int32(-32768)).astype(jnp.int16)
    lo = ((sk & jnp.int32(0xFFFF)) - jnp.int32(32768)).astype(jnp.int16)

    def _thresh16(cand):
        return (cand - jnp.int32(32768)).astype(jnp.int16)

    def body_hi(it, t):
        cand = t | (jnp.int32(1) << (15 - it))
        cnt = _count_true(hi >= _thresh16(cand))
        return jnp.where(cnt >= k_eff, cand, t)

    t_hi = jax.lax.fori_loop(0, 16, body_hi,
                             jnp.zeros((rows, 1), jnp.int32), unroll=8)
    h_s = _thresh16(t_hi)

    gt = hi > h_s
    eq = hi == h_s
    k_rem = k_eff - _count_true(gt)
    mlo = jnp.where(eq, lo, jnp.int16(-(2**15)))

    def body_lo(it, t):
        cand = t | (jnp.int32(1) << (15 - it))
        cnt = _count_true(mlo >= _thresh16(cand))
        return jnp.where(cnt >= k_rem, cand, t)

    t_lo = jax.lax.fori_loop(0, 16, body_lo,
                             jnp.zeros((rows, 1), jnp.int32), unroll=8)
    l_s = _thresh16(t_lo)

    m = gt | (eq & (lo >= l_s))
    if width < s:
        m = jnp.concatenate(
            [m, jnp.zeros((rows, s - width), jnp.bool_)], axis=1)
    mask_ref[0] = m


def _mask_kernel(k_fixed, nchunks, *refs):
    mask_ref = refs[-1]
    s = mask_ref.shape[2]
    j = pl.program_id(1)

    for i in range(nchunks):

        @pl.when(j == i)
        def _(i=i):
            _select_body(k_fixed, i, refs[i], mask_ref, s)


@functools.partial(jax.jit, static_argnames=())
def kernel(x, index_scores, Wq, Wk):
    bsz, s, _ = index_scores.shape
    k_fixed = min(_BASE_K, s)
    k_fixed = int(np.clip(k_fixed, _MIN_K, min(_MAX_K, s)))

    rows = min(_CHUNK, s)
    nchunks = s // rows
    grid = (bsz, nchunks)
    in_specs = [
        pl.BlockSpec((1, rows, (i + 1) * rows),
                     functools.partial(lambda i, b, r: (b, i, 0), i))
        for i in range(nchunks)
    ]
    mask = pl.pallas_call(
        functools.partial(_mask_kernel, k_fixed, nchunks),
        grid=grid,
        in_specs=in_specs,
        out_specs=pl.BlockSpec((1, rows, s), lambda b, r: (b, r, 0)),
        out_shape=jax.ShapeDtypeStruct((bsz, s, s), jnp.bool_),
        compiler_params=pltpu.CompilerParams(
            dimension_semantics=("parallel", "parallel")),
    )(*([index_scores] * nchunks))

    k_values = jnp.full((bsz, s), k_fixed, dtype=jnp.int32)
    return (mask, k_values)
